# Initial kernel scaffold; baseline (speedup 1.0000x reference)
import jax
import jax.numpy as jnp
from jax.experimental import pallas as pl


def kernel(x, edge_index, edge_weights, batch, Wm0, bm0, Wm1, bm1, Wl0, bl0, hop1, Wa1, ba1, Wb1, bb1, Wl1, bl1, hop2, Wa2, ba2, Wb2, bb2, Wl2, bl2):
    raise NotImplementedError("write your pallas kernel here")


# speedup vs baseline: 7.8150x; 7.8150x over previous
import functools

import jax
import jax.numpy as jnp
from jax import lax
from jax.experimental import pallas as pl
from jax.experimental.pallas import tpu as pltpu
from jax.experimental.pallas import tpu_sc as plsc

N = 10000
E = 320000
F_IN = 128
H = 64
C = 16
D = 5
G = 128

NC = 2
NS = 16
NW = NC * NS
CH = 125
EPW = E // NW
NCHUNK = EPW // CH
NBUF = 4
NP = 10240
RPT = NP // NS

_f32 = jnp.float32
_i32 = jnp.int32


def _bn(h):
    m = jnp.mean(h, axis=0, keepdims=True)
    v = jnp.mean((h - m) * (h - m), axis=0, keepdims=True)
    return (h - m) / jnp.sqrt(v + 1e-5)


BR = 1000


def _readout(logit_ref, batch_ref, out_prev):
    gids = lax.broadcasted_iota(jnp.int32, (1, G), 1)
    neg = jnp.float32(-jnp.inf)

    def blk(b, acc):
        lg = logit_ref[pl.ds(b * BR, BR), :]
        mask = batch_ref[pl.ds(b * BR, BR), :] == gids
        cols = []
        for c in range(C):
            mc = jnp.where(mask, lg[:, c:c + 1], neg)
            cols.append(jnp.max(mc, axis=0, keepdims=True))
        return jnp.maximum(acc, jnp.concatenate(cols, axis=0))

    acc = lax.fori_loop(0, N // BR, blk, jnp.full((C, G), neg, _f32))
    return out_prev + acc


def _head_body(x_ref, w0_ref, b0_ref, w1_ref, b1_ref, wl_ref, bl_ref,
               batch_ref, src_ref, ew_ref,
               h_ref, out_ref, gidx_ref, logit_scr):
    h = jax.nn.relu(_bn(x_ref[...] @ w0_ref[...] + b0_ref[...]))
    h = jax.nn.relu(_bn(h @ w1_ref[...] + b1_ref[...]))
    h_ref[...] = h
    logit_scr[...] = h @ wl_ref[...] + bl_ref[...]
    gidx_ref[...] = src_ref[...] + N * ew_ref[...]
    out_ref[...] = _readout(logit_scr, batch_ref, jnp.zeros((C, G), _f32))


def _layer_body(h_ref, p0_ref, p1_ref, wa_ref, ba_ref, wb_ref,
                bb_ref, wl_ref, bl_ref, batch_ref, outp_ref,
                h2_ref, out_ref, logit_scr):
    z = h_ref[...] + p0_ref[pl.ds(0, N), :] + p1_ref[pl.ds(0, N), :]
    z = jax.nn.relu(_bn(z @ wa_ref[...] + ba_ref[...]))
    h = jax.nn.relu(_bn(z @ wb_ref[...] + bb_ref[...]))
    h2_ref[...] = h
    logit_scr[...] = h @ wl_ref[...] + bl_ref[...]
    out_ref[...] = _readout(logit_scr, batch_ref, outp_ref[...])


def _table_body(h_ref, hop_ref, tab_ref):
    d = pl.program_id(0)
    hop_row = hop_ref[...]
    e = jnp.exp(hop_row - jnp.max(hop_row))
    coef = e / jnp.sum(e)
    dsel = lax.broadcasted_iota(jnp.int32, (1, D), 1) == d
    cd = jnp.sum(jnp.where(dsel, coef, 0.0))
    tab_ref[...] = cd * h_ref[...]


def _head_call(x, Wm0, bm0, Wm1, bm1, Wl0, bl0, batch2, src2, ew2):
    return pl.pallas_call(
        _head_body,
        out_shape=[
            jax.ShapeDtypeStruct((N, H), _f32),
            jax.ShapeDtypeStruct((C, G), _f32),
            jax.ShapeDtypeStruct((E // 128, 128), _i32),
        ],
        scratch_shapes=[pltpu.VMEM((N, C), _f32)],
    )(x, Wm0, bm0.reshape(1, H), Wm1, bm1.reshape(1, H), Wl0,
      bl0.reshape(1, C), batch2, src2, ew2)


def _layer_call(h, p0, p1, Wa, ba, Wb, bb, Wl, bl, batch2, out_prev):
    return pl.pallas_call(
        _layer_body,
        out_shape=[
            jax.ShapeDtypeStruct((N, H), _f32),
            jax.ShapeDtypeStruct((C, G), _f32),
        ],
        scratch_shapes=[pltpu.VMEM((N, C), _f32)],
    )(h, p0, p1, Wa, ba.reshape(1, H), Wb, bb.reshape(1, H), Wl,
      bl.reshape(1, C), batch2, out_prev)


def _table_call(h, hop):
    return pl.pallas_call(
        _table_body,
        grid=(D,),
        in_specs=[
            pl.BlockSpec((N, H), lambda d: (0, 0)),
            pl.BlockSpec((1, D), lambda d: (0, 0)),
        ],
        out_specs=pl.BlockSpec((N, H), lambda d: (d, 0)),
        out_shape=jax.ShapeDtypeStruct((D * N, H), _f32),
    )(h, hop.reshape(1, D))


def _sc_edge_agg(tab, gidx3, dst3):
    mesh = plsc.VectorSubcoreMesh(core_axis_name="c", subcore_axis_name="s")

    @functools.partial(
        pl.kernel,
        mesh=mesh,
        out_type=jax.ShapeDtypeStruct((NC, NP, H), _f32),
        compiler_params=pltpu.CompilerParams(use_tc_tiling_on_sc=False),
        scratch_types=[
            pltpu.VMEM((NCHUNK, CH), _i32),
            pltpu.VMEM((NCHUNK, CH), _i32),
            [pltpu.VMEM((CH, H), _f32) for _ in range(NBUF)],
            pltpu.VMEM((RPT // 5, H), _f32),
            pltpu.VMEM_SHARED((NP, H), _f32),
            [pltpu.SemaphoreType.DMA for _ in range(NBUF)],
            [pltpu.SemaphoreType.DMA for _ in range(NBUF)],
        ],
    )
    def k(tab_hbm, gidx_hbm, dst_hbm, out_hbm, idx_v, dst_v, rows, zero_v,
          acc_sh, gsems, ssems):
        cid = lax.axis_index("c")
        sid = lax.axis_index("s")
        wid = sid * NC + cid

        pltpu.sync_copy(gidx_hbm.at[wid], idx_v)
        pltpu.sync_copy(dst_hbm.at[wid], dst_v)

        def zbody(r, carry):
            for c4 in range(H // 16):
                zero_v[r, pl.ds(c4 * 16, 16)] = jnp.zeros((16,), _f32)
            return carry

        lax.fori_loop(0, RPT // 5, zbody, 0)
        for q in range(5):
            pltpu.sync_copy(
                zero_v, acc_sh.at[pl.ds(sid * RPT + q * (RPT // 5), RPT // 5)])
        plsc.subcore_barrier()

        def body(t, carry):
            j = t * NBUF
            gcps = []
            for b in range(NBUF):
                gcps.append(pltpu.async_copy(
                    tab_hbm.at[idx_v.at[j + b]], rows[b], gsems[b]))
            scps = []
            for b in range(NBUF):
                gcps[b].wait()
                scps.append(pltpu.async_copy(
                    rows[b], acc_sh.at[dst_v.at[j + b]], ssems[b], add=True))
            for b in range(NBUF):
                scps[b].wait()
            return carry

        lax.fori_loop(0, NCHUNK // NBUF, body, 0)
        plsc.subcore_barrier()
        pltpu.sync_copy(acc_sh.at[pl.ds(sid * RPT, RPT)],
                        out_hbm.at[cid, pl.ds(sid * RPT, RPT)])

    return k(tab, gidx3, dst3)


def kernel(x, edge_index, edge_weights, batch, Wm0, bm0, Wm1, bm1, Wl0, bl0,
           hop1, Wa1, ba1, Wb1, bb1, Wl1, bl1,
           hop2, Wa2, ba2, Wb2, bb2, Wl2, bl2):
    batch2 = batch.astype(_i32).reshape(N, 1)
    src2 = edge_index[0].astype(_i32).reshape(E // 128, 128)
    ew2 = edge_weights.astype(_i32).reshape(E // 128, 128)
    dst3 = edge_index[1].astype(_i32).reshape(NW, NCHUNK, CH)

    h0, out0, gidx2 = _head_call(x, Wm0, bm0, Wm1, bm1, Wl0, bl0,
                                 batch2, src2, ew2)
    gidx3 = gidx2.reshape(NW, NCHUNK, CH)
    tab1 = _table_call(h0, hop1)

    p1 = _sc_edge_agg(tab1, gidx3, dst3)
    h1, out1 = _layer_call(h0, p1[0], p1[1], Wa1, ba1, Wb1, bb1,
                           Wl1, bl1, batch2, out0)
    tab2 = _table_call(h1, hop2)

    p2 = _sc_edge_agg(tab2, gidx3, dst3)
    _, out2 = _layer_call(h1, p2[0], p2[1], Wa2, ba2, Wb2, bb2,
                          Wl2, bl2, batch2, out1)

    return out2.T


# speedup vs baseline: 10.4275x; 1.3343x over previous
import functools

import jax
import jax.numpy as jnp
from jax import lax
from jax.experimental import pallas as pl
from jax.experimental.pallas import tpu as pltpu
from jax.experimental.pallas import tpu_sc as plsc

N = 10000
E = 320000
F_IN = 128
H = 64
C = 16
D = 5
G = 128
NPAIR = N // 2

NC = 2
NS = 16
NW = NC * NS
CH = 125
EPW = E // NW
NCHUNK = EPW // CH
NBUF = 4
NP = 10240
RPT = NP // NS

_f32 = jnp.float32
_i32 = jnp.int32


def _blockdiag(w):
    a, b = w.shape
    z = jnp.zeros((a, b), _f32)
    return jnp.concatenate([jnp.concatenate([w, z], 1),
                            jnp.concatenate([z, w], 1)], 0)


def _bn_pair(hp):
    s1 = jnp.mean(hp, axis=0, keepdims=True)
    s2 = jnp.mean(hp * hp, axis=0, keepdims=True)
    half = s1.shape[1] // 2
    m = 0.5 * (s1[:, :half] + s1[:, half:])
    e2 = 0.5 * (s2[:, :half] + s2[:, half:])
    v = e2 - m * m
    m2 = jnp.concatenate([m, m], 1)
    v2 = jnp.concatenate([v, v], 1)
    return (hp - m2) / jnp.sqrt(v2 + 1e-5)


def _write_table(tab_ref, hop_row, hp):
    e = jnp.exp(hop_row - jnp.max(hop_row))
    coef = e / jnp.sum(e)
    for d in range(D):
        tab_ref[pl.ds(d * NPAIR, NPAIR), :] = coef[0, d] * hp


def _write_logits(lp_ref, hp, wlb, blb):
    lp_ref[pl.ds(0, NPAIR), :] = hp @ wlb + blb
    lp_ref[pl.ds(NPAIR, (NP - N) // 2), :] = jnp.zeros(
        ((NP - N) // 2, 2 * C), _f32)


def _head_body(xp_ref, w0_ref, b0_ref, w1_ref, b1_ref, wl_ref, bl_ref,
               hop_ref, src_ref, ew_ref,
               hp_ref, lp_ref, tab_ref, gidx_ref):
    w0b = _blockdiag(w0_ref[...])
    w1b = _blockdiag(w1_ref[...])
    b0b = jnp.concatenate([b0_ref[...], b0_ref[...]], 1)
    b1b = jnp.concatenate([b1_ref[...], b1_ref[...]], 1)
    hp = jax.nn.relu(_bn_pair(xp_ref[...] @ w0b + b0b))
    hp = jax.nn.relu(_bn_pair(hp @ w1b + b1b))
    hp_ref[...] = hp
    _write_logits(lp_ref, hp, _blockdiag(wl_ref[...]),
                  jnp.concatenate([bl_ref[...], bl_ref[...]], 1))
    _write_table(tab_ref, hop_ref[...], hp)
    gidx_ref[...] = src_ref[...] + N * ew_ref[...]


def _layer_body(with_table, hp_ref, p0_ref, p1_ref, wa_ref, ba_ref, wb_ref,
                bb_ref, wl_ref, bl_ref, hop_ref, *refs):
    if with_table:
        hp2_ref, lp_ref, tab_ref = refs
    else:
        hp2_ref, lp_ref = refs
    wab = _blockdiag(wa_ref[...])
    wbb = _blockdiag(wb_ref[...])
    bab = jnp.concatenate([ba_ref[...], ba_ref[...]], 1)
    bbb = jnp.concatenate([bb_ref[...], bb_ref[...]], 1)
    zp = hp_ref[...] + p0_ref[pl.ds(0, NPAIR), :] + p1_ref[pl.ds(0, NPAIR), :]
    zp = jax.nn.relu(_bn_pair(zp @ wab + bab))
    hp = jax.nn.relu(_bn_pair(zp @ wbb + bbb))
    hp2_ref[...] = hp
    _write_logits(lp_ref, hp, _blockdiag(wl_ref[...]),
                  jnp.concatenate([bl_ref[...], bl_ref[...]], 1))
    if with_table:
        _write_table(tab_ref, hop_ref[...], hp)


def _readout_body(lg_ref, batch_ref, outp_ref, out_ref):
    gids = lax.broadcasted_iota(jnp.int32, (1, G), 1)
    mask = batch_ref[...] == gids
    neg = jnp.float32(-jnp.inf)
    lg = lg_ref[...]
    cols = []
    for c in range(C):
        mc = jnp.where(mask, lg[:, c:c + 1], neg)
        cols.append(jnp.max(mc, axis=0, keepdims=True))
    out_ref[...] = outp_ref[...] + jnp.concatenate(cols, axis=0)


def _head_call(xp, Wm0, bm0, Wm1, bm1, Wl0, bl0, hop, src2, ew2):
    return pl.pallas_call(
        _head_body,
        out_shape=[
            jax.ShapeDtypeStruct((NPAIR, 2 * H), _f32),
            jax.ShapeDtypeStruct((NP // 2, 2 * C), _f32),
            jax.ShapeDtypeStruct((D * NPAIR, 2 * H), _f32),
            jax.ShapeDtypeStruct((E // 128, 128), _i32),
        ],
    )(xp, Wm0, bm0.reshape(1, H), Wm1, bm1.reshape(1, H), Wl0,
      bl0.reshape(1, C), hop.reshape(1, D), src2, ew2)


def _layer_call(with_table, hp, pp0, pp1, Wa, ba, Wb, bb, Wl, bl, hop):
    out_shape = [
        jax.ShapeDtypeStruct((NPAIR, 2 * H), _f32),
        jax.ShapeDtypeStruct((NP // 2, 2 * C), _f32),
    ]
    if with_table:
        out_shape.append(jax.ShapeDtypeStruct((D * NPAIR, 2 * H), _f32))
    return pl.pallas_call(
        functools.partial(_layer_body, with_table),
        out_shape=out_shape,
    )(hp, pp0, pp1, Wa, ba.reshape(1, H), Wb, bb.reshape(1, H), Wl,
      bl.reshape(1, C), hop.reshape(1, D))


def _readout_call(logits, batch2, out_prev):
    return pl.pallas_call(
        _readout_body,
        out_shape=jax.ShapeDtypeStruct((C, G), _f32),
    )(logits, batch2, out_prev)


def _sc_edge_agg(tab, gidx3, dst3):
    mesh = plsc.VectorSubcoreMesh(core_axis_name="c", subcore_axis_name="s")

    @functools.partial(
        pl.kernel,
        mesh=mesh,
        out_type=jax.ShapeDtypeStruct((NC, NP, H), _f32),
        compiler_params=pltpu.CompilerParams(use_tc_tiling_on_sc=False),
        scratch_types=[
            pltpu.VMEM((NCHUNK, CH), _i32),
            pltpu.VMEM((NCHUNK, CH), _i32),
            [pltpu.VMEM((CH, H), _f32) for _ in range(NBUF)],
            pltpu.VMEM((RPT // 5, H), _f32),
            pltpu.VMEM_SHARED((NP, H), _f32),
            [pltpu.SemaphoreType.DMA for _ in range(NBUF)],
            [pltpu.SemaphoreType.DMA for _ in range(NBUF)],
        ],
    )
    def k(tab_hbm, gidx_hbm, dst_hbm, out_hbm, idx_v, dst_v, rows, zero_v,
          acc_sh, gsems, ssems):
        cid = lax.axis_index("c")
        sid = lax.axis_index("s")
        wid = sid * NC + cid

        pltpu.sync_copy(gidx_hbm.at[wid], idx_v)
        pltpu.sync_copy(dst_hbm.at[wid], dst_v)

        def zbody(r, carry):
            for c4 in range(H // 16):
                zero_v[r, pl.ds(c4 * 16, 16)] = jnp.zeros((16,), _f32)
            return carry

        lax.fori_loop(0, RPT // 5, zbody, 0)
        for q in range(5):
            pltpu.sync_copy(
                zero_v, acc_sh.at[pl.ds(sid * RPT + q * (RPT // 5), RPT // 5)])
        plsc.subcore_barrier()

        def body(t, carry):
            j = t * NBUF
            gcps = []
            for b in range(NBUF):
                gcps.append(pltpu.async_copy(
                    tab_hbm.at[idx_v.at[j + b]], rows[b], gsems[b]))
            scps = []
            for b in range(NBUF):
                gcps[b].wait()
                scps.append(pltpu.async_copy(
                    rows[b], acc_sh.at[dst_v.at[j + b]], ssems[b], add=True))
            for b in range(NBUF):
                scps[b].wait()
            return carry

        lax.fori_loop(0, NCHUNK // NBUF, body, 0)
        plsc.subcore_barrier()
        pltpu.sync_copy(acc_sh.at[pl.ds(sid * RPT, RPT)],
                        out_hbm.at[cid, pl.ds(sid * RPT, RPT)])

    return k(tab, gidx3, dst3)


def kernel(x, edge_index, edge_weights, batch, Wm0, bm0, Wm1, bm1, Wl0, bl0,
           hop1, Wa1, ba1, Wb1, bb1, Wl1, bl1,
           hop2, Wa2, ba2, Wb2, bb2, Wl2, bl2):
    xp = x.reshape(NPAIR, 2 * F_IN)
    batch2 = jnp.concatenate(
        [batch.astype(_i32), jnp.full((NP - N,), G, _i32)]).reshape(NP, 1)
    src2 = edge_index[0].astype(_i32).reshape(E // 128, 128)
    ew2 = edge_weights.astype(_i32).reshape(E // 128, 128)
    dst3 = edge_index[1].astype(_i32).reshape(NW, NCHUNK, CH)

    hp0, lp0, tabp1, gidx2 = _head_call(xp, Wm0, bm0, Wm1, bm1, Wl0, bl0,
                                        hop1, src2, ew2)
    gidx3 = gidx2.reshape(NW, NCHUNK, CH)
    out0 = _readout_call(lp0.reshape(NP, C), batch2, jnp.zeros((C, G), _f32))

    p1 = _sc_edge_agg(tabp1.reshape(D * N, H), gidx3, dst3)
    pp1 = p1.reshape(NC, NP // 2, 2 * H)
    hp1, lp1, tabp2 = _layer_call(True, hp0, pp1[0], pp1[1], Wa1, ba1,
                                  Wb1, bb1, Wl1, bl1, hop2)
    out1 = _readout_call(lp1.reshape(NP, C), batch2, out0)

    p2 = _sc_edge_agg(tabp2.reshape(D * N, H), gidx3, dst3)
    pp2 = p2.reshape(NC, NP // 2, 2 * H)
    _, lp2 = _layer_call(False, hp1, pp2[0], pp2[1], Wa2, ba2, Wb2, bb2,
                         Wl2, bl2, hop2)
    out2 = _readout_call(lp2.reshape(NP, C), batch2, out1)

    return out2.T


# speedup vs baseline: 10.5597x; 1.0127x over previous
import functools

import jax
import jax.numpy as jnp
from jax import lax
from jax.experimental import pallas as pl
from jax.experimental.pallas import tpu as pltpu
from jax.experimental.pallas import tpu_sc as plsc

N = 10000
E = 320000
F_IN = 128
H = 64
C = 16
D = 5
G = 128
NPAIR = N // 2

NC = 2
NS = 16
NW = NC * NS
CH = 125
EPW = E // NW
NCHUNK = EPW // CH
NBUF = 4
NP = 10240
RPT = NP // NS

_f32 = jnp.float32
_i32 = jnp.int32


def _blockdiag(w):
    a, b = w.shape
    z = jnp.zeros((a, b), _f32)
    return jnp.concatenate([jnp.concatenate([w, z], 1),
                            jnp.concatenate([z, w], 1)], 0)


def _bn_pair(hp):
    s1 = jnp.mean(hp, axis=0, keepdims=True)
    s2 = jnp.mean(hp * hp, axis=0, keepdims=True)
    half = s1.shape[1] // 2
    m = 0.5 * (s1[:, :half] + s1[:, half:])
    e2 = 0.5 * (s2[:, :half] + s2[:, half:])
    v = e2 - m * m
    m2 = jnp.concatenate([m, m], 1)
    v2 = jnp.concatenate([v, v], 1)
    return (hp - m2) / jnp.sqrt(v2 + 1e-5)


def _write_table(tab_ref, hop_row, hp):
    e = jnp.exp(hop_row - jnp.max(hop_row))
    coef = e / jnp.sum(e)
    for d in range(D):
        tab_ref[pl.ds(d * NPAIR, NPAIR), :] = coef[0, d] * hp


def _write_logits(lp_ref, hp, wlb, blb):
    lp_ref[pl.ds(0, NPAIR), :] = hp @ wlb + blb
    lp_ref[pl.ds(NPAIR, (NP - N) // 2), :] = jnp.zeros(
        ((NP - N) // 2, 2 * C), _f32)


def _head_body(xp_ref, w0_ref, b0_ref, w1_ref, b1_ref, wl_ref, bl_ref,
               hop_ref, src_ref, ew_ref, bp_ref,
               hp_ref, lp_ref, tab_ref, gidx_ref, idxp_ref):
    w0b = _blockdiag(w0_ref[...])
    w1b = _blockdiag(w1_ref[...])
    b0b = jnp.concatenate([b0_ref[...], b0_ref[...]], 1)
    b1b = jnp.concatenate([b1_ref[...], b1_ref[...]], 1)
    hp = jax.nn.relu(_bn_pair(xp_ref[...] @ w0b + b0b))
    hp = jax.nn.relu(_bn_pair(hp @ w1b + b1b))
    hp_ref[...] = hp
    _write_logits(lp_ref, hp, _blockdiag(wl_ref[...]),
                  jnp.concatenate([bl_ref[...], bl_ref[...]], 1))
    _write_table(tab_ref, hop_ref[...], hp)
    gidx_ref[...] = src_ref[...] + N * ew_ref[...]
    ci = lax.broadcasted_iota(_i32, (1, C), 1)
    bp = bp_ref[...]
    idxp_ref[...] = jnp.concatenate(
        [bp[:, 0:1] * C + ci, bp[:, 1:2] * C + ci], 1)


def _layer_body(with_table, hp_ref, p0_ref, p1_ref, wa_ref, ba_ref, wb_ref,
                bb_ref, wl_ref, bl_ref, hop_ref, *refs):
    if with_table:
        hp2_ref, lp_ref, tab_ref = refs
    else:
        hp2_ref, lp_ref = refs
    wab = _blockdiag(wa_ref[...])
    wbb = _blockdiag(wb_ref[...])
    bab = jnp.concatenate([ba_ref[...], ba_ref[...]], 1)
    bbb = jnp.concatenate([bb_ref[...], bb_ref[...]], 1)
    zp = hp_ref[...] + p0_ref[pl.ds(0, NPAIR), :] + p1_ref[pl.ds(0, NPAIR), :]
    zp = jax.nn.relu(_bn_pair(zp @ wab + bab))
    hp = jax.nn.relu(_bn_pair(zp @ wbb + bbb))
    hp2_ref[...] = hp
    _write_logits(lp_ref, hp, _blockdiag(wl_ref[...]),
                  jnp.concatenate([bl_ref[...], bl_ref[...]], 1))
    if with_table:
        _write_table(tab_ref, hop_ref[...], hp)


def _combine_body(r1_ref, r2_ref, r3_ref, out_ref):
    out_ref[...] = (jnp.max(r1_ref[...], axis=0) + jnp.max(r2_ref[...], axis=0)
                    + jnp.max(r3_ref[...], axis=0))


RPW = NP // NW
GP = G + 8


def _sc_readout(lg_hbm, idx_hbm, ro_hbm, wid, lg_v, im_v, lo_v):
    pltpu.sync_copy(lg_hbm.at[pl.ds(wid * RPW, RPW)], lg_v)
    pltpu.sync_copy(idx_hbm.at[pl.ds(wid * RPW, RPW)], im_v)
    neg = jnp.full((16,), -jnp.inf, _f32)

    def ibody(r, carry):
        lo_v[pl.ds(r * 16, 16)] = neg
        return carry

    lax.fori_loop(0, GP * C // 16, ibody, 0)

    def rbody(r, carry):
        iv = im_v[r, pl.ds(0, C)]
        cur = plsc.load_gather(lo_v, [iv])
        mx = jnp.maximum(cur, lg_v[r, pl.ds(0, C)])
        plsc.store_scatter(lo_v, [iv], mx)
        return carry

    lax.fori_loop(0, RPW, rbody, 0)
    pltpu.sync_copy(lo_v.at[pl.ds(0, G * C)], ro_hbm.at[wid])


def _sc_ro_only(lg, idxm):
    mesh = plsc.VectorSubcoreMesh(core_axis_name="c", subcore_axis_name="s")

    @functools.partial(
        pl.kernel,
        mesh=mesh,
        out_type=jax.ShapeDtypeStruct((NW, G * C), _f32),
        compiler_params=pltpu.CompilerParams(use_tc_tiling_on_sc=False,
                                            needs_layout_passes=False),
        scratch_types=[
            pltpu.VMEM((RPW, C), _f32),
            pltpu.VMEM((RPW, C), _i32),
            pltpu.VMEM((GP * C,), _f32),
        ],
    )
    def k(lg_hbm, idx_hbm, ro_hbm, lg_v, im_v, lo_v):
        cid = lax.axis_index("c")
        sid = lax.axis_index("s")
        wid = sid * NC + cid
        _sc_readout(lg_hbm, idx_hbm, ro_hbm, wid, lg_v, im_v, lo_v)

    return k(lg, idxm)


def _head_call(xp, Wm0, bm0, Wm1, bm1, Wl0, bl0, hop, src2, ew2, bp2):
    return pl.pallas_call(
        _head_body,
        out_shape=[
            jax.ShapeDtypeStruct((NPAIR, 2 * H), _f32),
            jax.ShapeDtypeStruct((NP // 2, 2 * C), _f32),
            jax.ShapeDtypeStruct((D * NPAIR, 2 * H), _f32),
            jax.ShapeDtypeStruct((E // 128, 128), _i32),
            jax.ShapeDtypeStruct((NP // 2, 2 * C), _i32),
        ],
    )(xp, Wm0, bm0.reshape(1, H), Wm1, bm1.reshape(1, H), Wl0,
      bl0.reshape(1, C), hop.reshape(1, D), src2, ew2, bp2)


def _layer_call(with_table, hp, pp0, pp1, Wa, ba, Wb, bb, Wl, bl, hop):
    out_shape = [
        jax.ShapeDtypeStruct((NPAIR, 2 * H), _f32),
        jax.ShapeDtypeStruct((NP // 2, 2 * C), _f32),
    ]
    if with_table:
        out_shape.append(jax.ShapeDtypeStruct((D * NPAIR, 2 * H), _f32))
    return pl.pallas_call(
        functools.partial(_layer_body, with_table),
        out_shape=out_shape,
    )(hp, pp0, pp1, Wa, ba.reshape(1, H), Wb, bb.reshape(1, H), Wl,
      bl.reshape(1, C), hop.reshape(1, D))


def _combine_call(r1, r2, r3):
    return pl.pallas_call(
        _combine_body,
        out_shape=jax.ShapeDtypeStruct((G, C), _f32),
    )(r1.reshape(NW, G, C), r2.reshape(NW, G, C), r3.reshape(NW, G, C))


def _sc_edge_agg(tab, gidx3, dst3, lg, idxm):
    mesh = plsc.VectorSubcoreMesh(core_axis_name="c", subcore_axis_name="s")

    @functools.partial(
        pl.kernel,
        mesh=mesh,
        out_type=[jax.ShapeDtypeStruct((NC, NP, H), _f32),
                  jax.ShapeDtypeStruct((NW, G * C), _f32)],
        compiler_params=pltpu.CompilerParams(use_tc_tiling_on_sc=False,
                                            needs_layout_passes=False),
        scratch_types=[
            pltpu.VMEM((NCHUNK, CH), _i32),
            pltpu.VMEM((NCHUNK, CH), _i32),
            [pltpu.VMEM((CH, H), _f32) for _ in range(NBUF)],
            pltpu.VMEM((RPT // 5, H), _f32),
            pltpu.VMEM((RPW, C), _f32),
            pltpu.VMEM((RPW, C), _i32),
            pltpu.VMEM((GP * C,), _f32),
            pltpu.VMEM_SHARED((NP, H), _f32),
            [pltpu.SemaphoreType.DMA for _ in range(NBUF)],
            [pltpu.SemaphoreType.DMA for _ in range(NBUF)],
        ],
    )
    def k(tab_hbm, gidx_hbm, dst_hbm, lg_hbm, idxm_hbm, out_hbm, ro_hbm,
          idx_v, dst_v, rows, zero_v, lg_v, im_v, lo_v, acc_sh, gsems,
          ssems):
        cid = lax.axis_index("c")
        sid = lax.axis_index("s")
        wid = sid * NC + cid

        pltpu.sync_copy(gidx_hbm.at[wid], idx_v)
        pltpu.sync_copy(dst_hbm.at[wid], dst_v)

        _sc_readout(lg_hbm, idxm_hbm, ro_hbm, wid, lg_v, im_v, lo_v)

        def zbody(r, carry):
            for c4 in range(H // 16):
                zero_v[r, pl.ds(c4 * 16, 16)] = jnp.zeros((16,), _f32)
            return carry

        lax.fori_loop(0, RPT // 5, zbody, 0)
        for q in range(5):
            pltpu.sync_copy(
                zero_v, acc_sh.at[pl.ds(sid * RPT + q * (RPT // 5), RPT // 5)])
        plsc.subcore_barrier()

        def body(t, carry):
            j = t * NBUF
            gcps = []
            for b in range(NBUF):
                gcps.append(pltpu.async_copy(
                    tab_hbm.at[idx_v.at[j + b]], rows[b], gsems[b]))
            scps = []
            for b in range(NBUF):
                gcps[b].wait()
                scps.append(pltpu.async_copy(
                    rows[b], acc_sh.at[dst_v.at[j + b]], ssems[b], add=True))
            for b in range(NBUF):
                scps[b].wait()
            return carry

        lax.fori_loop(0, NCHUNK // NBUF, body, 0)
        plsc.subcore_barrier()
        pltpu.sync_copy(acc_sh.at[pl.ds(sid * RPT, RPT)],
                        out_hbm.at[cid, pl.ds(sid * RPT, RPT)])

    return k(tab, gidx3, dst3, lg, idxm)


def kernel(x, edge_index, edge_weights, batch, Wm0, bm0, Wm1, bm1, Wl0, bl0,
           hop1, Wa1, ba1, Wb1, bb1, Wl1, bl1,
           hop2, Wa2, ba2, Wb2, bb2, Wl2, bl2):
    xp = x.reshape(NPAIR, 2 * F_IN)
    batch_pad = jnp.concatenate(
        [batch.astype(_i32), jnp.full((NP - N,), G, _i32)])
    src2 = edge_index[0].astype(_i32).reshape(E // 128, 128)
    ew2 = edge_weights.astype(_i32).reshape(E // 128, 128)
    dst3 = edge_index[1].astype(_i32).reshape(NW, NCHUNK, CH)

    bp2 = batch_pad.reshape(NP // 2, 2)
    hp0, lp0, tabp1, gidx2, idxp = _head_call(xp, Wm0, bm0, Wm1, bm1, Wl0,
                                              bl0, hop1, src2, ew2, bp2)
    gidx3 = gidx2.reshape(NW, NCHUNK, CH)
    idxm = idxp.reshape(NP, C)

    p1, ro1 = _sc_edge_agg(tabp1.reshape(D * N, H), gidx3, dst3,
                           lp0.reshape(NP, C), idxm)
    pp1 = p1.reshape(NC, NP // 2, 2 * H)
    hp1, lp1, tabp2 = _layer_call(True, hp0, pp1[0], pp1[1], Wa1, ba1,
                                  Wb1, bb1, Wl1, bl1, hop2)

    p2, ro2 = _sc_edge_agg(tabp2.reshape(D * N, H), gidx3, dst3,
                           lp1.reshape(NP, C), idxm)
    pp2 = p2.reshape(NC, NP // 2, 2 * H)
    _, lp2 = _layer_call(False, hp1, pp2[0], pp2[1], Wa2, ba2, Wb2, bb2,
                         Wl2, bl2, hop2)

    ro3 = _sc_ro_only(lp2.reshape(NP, C), idxm)
    return _combine_call(ro1, ro2, ro3)
